# baseline (speedup 1.0000x reference)
import functools

import jax
import jax.numpy as jnp
from jax import lax
from jax.experimental import pallas as pl
from jax.experimental.pallas import tpu as pltpu
from jax.experimental.pallas import tpu_sc as plsc

NC = 2
NS = 16
L = 16
C = 48
WC = 32


def _tc_body(x_ref, w1_ref, b1_ref, w2_ref, b2_ref, wsd_ref, bsd_ref,
             feat_ref, a_ref):
    xb = x_ref[...]
    h1 = jnp.maximum(
        jnp.dot(xb, w1_ref[0], preferred_element_type=jnp.float32) + b1_ref[0],
        0.0)
    ft = jnp.dot(h1, w2_ref[0], preferred_element_type=jnp.float32) + b2_ref[0]
    feat_ref[0] = ft
    a_ref[0] = jnp.dot(ft, wsd_ref[0], preferred_element_type=jnp.float32) + bsd_ref[0]


def _tc_stage(x, W1, b1, W2, b2, wsd, bsd):
    N, D_IN = x.shape
    H, _, D_HID = W1.shape
    D_OUT = W2.shape[2]
    B = N // 10 if N % 10 == 0 else N
    NB = N // B
    b1r = b1[:, None, :]
    b2r = b2[:, None, :]
    return pl.pallas_call(
        _tc_body,
        grid=(H, NB),
        in_specs=[
            pl.BlockSpec((B, D_IN), lambda h, i: (i, 0)),
            pl.BlockSpec((1, D_IN, D_HID), lambda h, i: (h, 0, 0)),
            pl.BlockSpec((1, 1, D_HID), lambda h, i: (h, 0, 0)),
            pl.BlockSpec((1, D_HID, D_OUT), lambda h, i: (h, 0, 0)),
            pl.BlockSpec((1, 1, D_OUT), lambda h, i: (h, 0, 0)),
            pl.BlockSpec((1, D_OUT, 2), lambda h, i: (h, 0, 0)),
            pl.BlockSpec((1, 1, 2), lambda h, i: (h, 0, 0)),
        ],
        out_specs=[
            pl.BlockSpec((1, B, D_OUT), lambda h, i: (h, i, 0)),
            pl.BlockSpec((1, B, 2), lambda h, i: (h, i, 0)),
        ],
        out_shape=[
            jax.ShapeDtypeStruct((H, N, D_OUT), jnp.float32),
            jax.ShapeDtypeStruct((H, N, 2), jnp.float32),
        ],
    )(x, W1, b1r, W2, b2r, wsd, bsd)


def _sc_stage(src, pk, apk, wae16, featf, aridx, zerosf,
              NP, NPS, D_OUT, EPT):
    H = apk.shape[0] // NPS
    HNP = H * NP
    NH = featf.shape[0] // H
    NPA = NPS + 8
    NCH = EPT // C
    RPT = NP // NS
    NZC = RPT // WC
    mesh = plsc.VectorSubcoreMesh(core_axis_name="c", subcore_axis_name="s")

    @functools.partial(
        pl.kernel, mesh=mesh,
        compiler_params=pltpu.CompilerParams(needs_layout_passes=False),
        out_type=[
            jax.ShapeDtypeStruct((HNP, D_OUT), jnp.float32),
            jax.ShapeDtypeStruct((H * NS * NPS,), jnp.float32),
            jax.ShapeDtypeStruct((H * NS * NPS,), jnp.float32),
        ],
        scratch_types=[
            pltpu.VMEM((NPS,), jnp.float32),
            pltpu.VMEM((NPS,), jnp.float32),
            pltpu.VMEM((NPS,), jnp.float32),
            pltpu.VMEM((L,), jnp.float32),
            pltpu.VMEM((3, C), jnp.int32),
            pltpu.VMEM((3, C), jnp.int32),
            pltpu.VMEM((C,), jnp.int32),
            pltpu.VMEM((WC,), jnp.int32),
            pltpu.VMEM((C,), jnp.float32),
            pltpu.VMEM((C, 128), jnp.float32),
            pltpu.VMEM((C, 128), jnp.float32),
            pltpu.VMEM((C, 128), jnp.float32),
            pltpu.VMEM_SHARED((NPA, 128), jnp.float32),
            pltpu.SemaphoreType.DMA,
            pltpu.SemaphoreType.DMA,
            pltpu.SemaphoreType.DMA,
            pltpu.SemaphoreType.DMA,
            pltpu.SemaphoreType.DMA,
        ],
    )
    def sc_kernel(src_h, pk_h, apk_h, wae_h, feat_h,
                  aridx_h, zeros_h, accd_h, rsp_h, esp_h,
                  apk_v, rs_v, es_v, wae_v, pkA_v, pkB_v, sidxp_v,
                  zidx_v, elem_v, rowsA_v, rowsB_v, rowsC_v, acc_sh,
                  sem, sem2, sem3, sem4, sem5):
        c = lax.axis_index("c")
        s = lax.axis_index("s")
        hoff = c * NP

        zf = jnp.zeros((L,), jnp.float32)

        for j in range(C):
            for k in range(128 // L):
                rowsB_v[j, pl.ds(k * L, L)] = zf

        pltpu.sync_copy(apk_h.at[pl.ds(pl.multiple_of(c * NPS, 8), NPS)],
                        apk_v)
        pltpu.sync_copy(wae_h.at[pl.ds(c * L, L)], wae_v)
        pltpu.sync_copy(zeros_h, rs_v)
        pltpu.sync_copy(zeros_h, es_v)

        def zero_body(i, _):
            base = s * RPT + i * WC
            pltpu.sync_copy(aridx_h.at[pl.ds(base, WC)], zidx_v)
            pltpu.sync_copy(rowsB_v.at[pl.ds(0, WC)], acc_sh.at[zidx_v])
            return 0

        lax.fori_loop(0, NZC, zero_body, 0)
        plsc.subcore_barrier()

        base_e = s * EPT
        pkbase = (c * NS + s) * NCH
        wae = wae_v[...]

        hN = c * NH

        def compute_elem(pk_v):
            for g in range(C // L):
                sl = pl.ds(g * L, L)
                si = pk_v[0, sl]
                di = pk_v[1, sl] - hN
                g1 = plsc.bitcast(plsc.load_gather(apk_v, [si]), jnp.int32)
                g2 = plsc.bitcast(plsc.load_gather(apk_v, [di]), jnp.int32)
                a1 = plsc.bitcast(g1 << 16, jnp.float32)
                a2 = plsc.bitcast(g2 & jnp.int32(-65536), jnp.float32)
                evg = plsc.bitcast(pk_v[2, sl], jnp.float32)
                sc = a1 + a2 + evg * wae
                lr = jnp.where(sc > 0.0, sc, sc * 0.01)
                el = jnp.exp(-lr)
                elem_v[sl] = el
                plsc.addupdate_scatter(rs_v, [si], el)
                plsc.addupdate_scatter(es_v, [si], el * evg)

        def scale(src_rows, dst_rows):
            for r in range(C):
                b = plsc.load_gather(elem_v, [jnp.full((L,), r, jnp.int32)])
                for k in range(128 // L):
                    sl2 = pl.ds(k * L, L)
                    dst_rows[r, sl2] = src_rows[r, sl2] * b

        def pair_body(i2, _):
            a = 2 * i2
            pklA = pltpu.async_copy(pk_h.at[pkbase + a], pkA_v, sem3)
            pklB = pltpu.async_copy(pk_h.at[pkbase + a + 1], pkB_v, sem4)
            pklA.wait()
            gA = pltpu.async_copy(feat_h.at[pkA_v.at[1]], rowsA_v, sem)
            pklB.wait()
            gB = pltpu.async_copy(feat_h.at[pkB_v.at[1]], rowsC_v, sem5)
            spoff = base_e + jnp.maximum(a - 1, 0) * C
            pltpu.async_copy(src_h.at[pl.ds(spoff, C)], sidxp_v, sem4).wait()
            scat1 = pltpu.async_copy(rowsB_v, acc_sh.at[sidxp_v], sem2,
                                     add=True)
            compute_elem(pkA_v)
            gA.wait()
            scat1.wait()
            scale(rowsA_v, rowsA_v)
            pltpu.async_copy(src_h.at[pl.ds(base_e + a * C, C)], sidxp_v,
                             sem4).wait()
            compute_elem(pkB_v)
            scat2 = pltpu.async_copy(rowsA_v, acc_sh.at[sidxp_v], sem2,
                                     add=True)
            gB.wait()
            scale(rowsC_v, rowsB_v)
            scat2.wait()
            return 0

        lax.fori_loop(0, NCH // 2, pair_body, 0)
        poff = pl.multiple_of((c * NS + s) * NPS, 8)
        pltpu.sync_copy(rs_v, rsp_h.at[pl.ds(poff, NPS)])
        pltpu.sync_copy(es_v, esp_h.at[pl.ds(poff, NPS)])
        pltpu.sync_copy(src_h.at[pl.ds(base_e + (NCH - 1) * C, C)], sidxp_v)
        pltpu.sync_copy(rowsB_v, acc_sh.at[sidxp_v], add=True)
        plsc.subcore_barrier()

        def out_body(i, _):
            base = s * RPT + i * WC
            pltpu.sync_copy(aridx_h.at[pl.ds(base, WC)], zidx_v)
            pltpu.sync_copy(acc_sh.at[zidx_v], rowsA_v.at[pl.ds(0, WC)])
            pltpu.sync_copy(rowsA_v.at[pl.ds(0, WC)],
                            accd_h.at[pl.ds(hoff + base, WC)])
            return 0

        lax.fori_loop(0, NZC, out_body, 0)

    return sc_kernel(src, pk, apk, wae16, featf, aridx, zerosf)


def _tc3_body(feat_ref, acc_ref, rspt_ref, espt_ref, out_ref):
    H = feat_ref.shape[0]
    pieces = []
    for h in range(H):
        rs = jnp.sum(rspt_ref[h], axis=-1)
        es = jnp.sum(espt_ref[h], axis=-1)
        rcp = 1.0 / rs
        pieces += [feat_ref[h], acc_ref[h] * rcp[:, None],
                   (es * rcp)[:, None]]
    out_ref[...] = jnp.concatenate(pieces, axis=1)


def _tc3_stage(feat, accd, rspt, espt, N):
    H, NP, D = accd.shape
    BS = N // 10 if N % 10 == 0 else N
    NB = N // BS
    return pl.pallas_call(
        _tc3_body,
        grid=(NB,),
        in_specs=[
            pl.BlockSpec((H, BS, D), lambda i: (0, i, 0)),
            pl.BlockSpec((H, BS, D), lambda i: (0, i, 0)),
            pl.BlockSpec((H, BS, NS), lambda i: (0, i, 0)),
            pl.BlockSpec((H, BS, NS), lambda i: (0, i, 0)),
        ],
        out_specs=pl.BlockSpec((BS, H * (2 * D + 1)), lambda i: (i, 0)),
        out_shape=jax.ShapeDtypeStruct((N, H * (2 * D + 1)), jnp.float32),
    )(feat, accd, rspt, espt)


def kernel(x, edge_index, edge_values, num_nodes, W1, b1, W2, b2, Wa, ba):
    N, D_IN = x.shape
    H = W1.shape[0]
    D_OUT = W2.shape[2]
    E = edge_index.shape[1]

    wsd = jnp.stack([Wa[:, :D_OUT, 0], Wa[:, D_OUT:2 * D_OUT, 0]], axis=-1) * 0.05
    bsd = jnp.concatenate([ba * 0.05, jnp.zeros_like(ba)], axis=1)[:, None, :]
    wae16 = jnp.repeat(Wa[:, 2 * D_OUT, 0] * 0.05, L).reshape(H * L)

    feat, a2 = _tc_stage(x, W1, b1, W2, b2, wsd, bsd)

    NP = -(-(N + 1) // (NS * WC)) * (NS * WC)
    NPS = -(-(N + 1) // 8) * 8
    featp = feat.reshape(H * N, D_OUT)
    ap = jnp.pad(a2, ((0, 0), (0, NPS - N), (0, 0)))
    asr16 = lax.bitcast_convert_type(
        ap[:, :, 0].astype(jnp.bfloat16), jnp.uint16).astype(jnp.uint32)
    adt16 = lax.bitcast_convert_type(
        ap[:, :, 1].astype(jnp.bfloat16), jnp.uint16).astype(jnp.uint32)
    apk = lax.bitcast_convert_type(
        (adt16 << 16) | asr16, jnp.float32).reshape(H * NPS)

    EPT = -(-E // (NS * 2 * C)) * (2 * C)
    EPAD = EPT * NS - E
    src = edge_index[0]
    dst = edge_index[1]
    ev = edge_values
    if EPAD:
        src = jnp.concatenate([src, jnp.full((EPAD,), N, jnp.int32)])
        dst = jnp.concatenate([dst, jnp.zeros((EPAD,), jnp.int32)])
        ev = jnp.concatenate([ev, jnp.zeros((EPAD,), jnp.float32)])
    NCH = EPT // C
    dsta = dst[None, :] + (jnp.arange(H, dtype=jnp.int32) * N)[:, None]
    src_r = jnp.broadcast_to(src.reshape(1, NS, NCH, C), (H, NS, NCH, C))
    ev_r = jnp.broadcast_to(
        lax.bitcast_convert_type(ev, jnp.int32).reshape(1, NS, NCH, C),
        (H, NS, NCH, C))
    pk = jnp.stack([src_r, dsta.reshape(H, NS, NCH, C), ev_r],
                   axis=3).reshape(H * NS * NCH, 3, C)

    aridx = jnp.minimum(jnp.arange(NP, dtype=jnp.int32),
                        jnp.int32(NPS + 7))
    zerosf = jnp.zeros((NPS,), jnp.float32)
    accd, rsp, esp = _sc_stage(src, pk, apk, wae16, featp,
                               aridx, zerosf, NP, NPS, D_OUT, EPT)
    rspt = rsp.reshape(H, NS, NPS).transpose(0, 2, 1)
    espt = esp.reshape(H, NS, NPS).transpose(0, 2, 1)
    return _tc3_stage(feat, accd.reshape(H, NP, D_OUT), rspt, espt, N)


# speedup vs baseline: 1.2494x; 1.2494x over previous
import functools

import jax
import jax.numpy as jnp
from jax import lax
from jax.experimental import pallas as pl
from jax.experimental.pallas import tpu as pltpu
from jax.experimental.pallas import tpu_sc as plsc

NC = 2
NS = 16
L = 16
C = 64
WC = 64


def _tc_body(x_ref, w1_ref, b1_ref, w2_ref, b2_ref, wsd_ref, bsd_ref,
             feat_ref, a_ref):
    xb = x_ref[...]
    h1 = jnp.maximum(
        jnp.dot(xb, w1_ref[0], preferred_element_type=jnp.float32) + b1_ref[0],
        0.0)
    ft = jnp.dot(h1, w2_ref[0], preferred_element_type=jnp.float32) + b2_ref[0]
    feat_ref[0] = ft
    a_ref[0] = jnp.dot(ft, wsd_ref[0], preferred_element_type=jnp.float32) + bsd_ref[0]


def _tc_stage(x, W1, b1, W2, b2, wsd, bsd):
    N, D_IN = x.shape
    H, _, D_HID = W1.shape
    D_OUT = W2.shape[2]
    B = N // 10 if N % 10 == 0 else N
    NB = N // B
    b1r = b1[:, None, :]
    b2r = b2[:, None, :]
    return pl.pallas_call(
        _tc_body,
        grid=(H, NB),
        in_specs=[
            pl.BlockSpec((B, D_IN), lambda h, i: (i, 0)),
            pl.BlockSpec((1, D_IN, D_HID), lambda h, i: (h, 0, 0)),
            pl.BlockSpec((1, 1, D_HID), lambda h, i: (h, 0, 0)),
            pl.BlockSpec((1, D_HID, D_OUT), lambda h, i: (h, 0, 0)),
            pl.BlockSpec((1, 1, D_OUT), lambda h, i: (h, 0, 0)),
            pl.BlockSpec((1, D_OUT, 2), lambda h, i: (h, 0, 0)),
            pl.BlockSpec((1, 1, 2), lambda h, i: (h, 0, 0)),
        ],
        out_specs=[
            pl.BlockSpec((1, B, D_OUT), lambda h, i: (h, i, 0)),
            pl.BlockSpec((1, B, 2), lambda h, i: (h, i, 0)),
        ],
        out_shape=[
            jax.ShapeDtypeStruct((H, N, D_OUT), jnp.float32),
            jax.ShapeDtypeStruct((H, N, 2), jnp.float32),
        ],
    )(x, W1, b1r, W2, b2r, wsd, bsd)


def _sc_stage(src, pk, apk, wae16, featf, aridx, zerosf,
              NP, NPS, D_OUT, EPT):
    H = apk.shape[0] // NPS
    HNP = H * NP
    NH = featf.shape[0] // H
    NCH = EPT // C
    RPT = NP // NS
    NZC = RPT // WC
    mesh = plsc.VectorSubcoreMesh(core_axis_name="c", subcore_axis_name="s")

    @functools.partial(
        pl.kernel, mesh=mesh,
        compiler_params=pltpu.CompilerParams(needs_layout_passes=False),
        out_type=[
            jax.ShapeDtypeStruct((HNP, D_OUT), jnp.float32),
            jax.ShapeDtypeStruct((H * NS * NPS,), jnp.float32),
            jax.ShapeDtypeStruct((H * NS * NPS,), jnp.float32),
        ],
        scratch_types=[
            pltpu.VMEM((NPS,), jnp.float32),
            pltpu.VMEM((NPS,), jnp.float32),
            pltpu.VMEM((NPS,), jnp.float32),
            pltpu.VMEM((L,), jnp.float32),
            pltpu.VMEM((3, C), jnp.int32),
            pltpu.VMEM((C,), jnp.int32),
            pltpu.VMEM((WC,), jnp.int32),
            pltpu.VMEM((C,), jnp.float32),
            pltpu.VMEM((C, 128), jnp.float32),
            pltpu.VMEM((C, 128), jnp.float32),
            pltpu.VMEM_SHARED((NP, 128), jnp.float32),
            pltpu.SemaphoreType.DMA,
            pltpu.SemaphoreType.DMA,
            pltpu.SemaphoreType.DMA,
            pltpu.SemaphoreType.DMA,
        ],
    )
    def sc_kernel(src_h, pk_h, apk_h, wae_h, feat_h,
                  aridx_h, zeros_h, accd_h, rsp_h, esp_h,
                  apk_v, rs_v, es_v, wae_v, pkA_v, sidxp_v,
                  zidx_v, elem_v, rowsA_v, rowsB_v, acc_sh,
                  sem, sem2, sem3, sem4):
        c = lax.axis_index("c")
        s = lax.axis_index("s")
        hoff = c * NP

        zf = jnp.zeros((L,), jnp.float32)

        for j in range(C):
            for k in range(128 // L):
                rowsB_v[j, pl.ds(k * L, L)] = zf

        pltpu.sync_copy(apk_h.at[pl.ds(pl.multiple_of(c * NPS, 8), NPS)],
                        apk_v)
        pltpu.sync_copy(wae_h.at[pl.ds(c * L, L)], wae_v)
        pltpu.sync_copy(zeros_h, rs_v)
        pltpu.sync_copy(zeros_h, es_v)

        def zero_body(i, _):
            base = s * RPT + i * WC
            pltpu.sync_copy(aridx_h.at[pl.ds(base, WC)], zidx_v)
            pltpu.sync_copy(rowsB_v.at[pl.ds(0, WC)], acc_sh.at[zidx_v])
            return 0

        lax.fori_loop(0, NZC, zero_body, 0)
        plsc.subcore_barrier()

        base_e = s * EPT
        pkbase = (c * NS + s) * NCH
        wae = wae_v[...]

        hN = c * NH

        def compute_elem(pk_v):
            for g in range(C // L):
                sl = pl.ds(g * L, L)
                si = pk_v[0, sl]
                di = pk_v[1, sl] - hN
                g1 = plsc.bitcast(plsc.load_gather(apk_v, [si]), jnp.int32)
                g2 = plsc.bitcast(plsc.load_gather(apk_v, [di]), jnp.int32)
                a1 = plsc.bitcast(g1 << 16, jnp.float32)
                a2 = plsc.bitcast(g2 & jnp.int32(-65536), jnp.float32)
                evg = plsc.bitcast(pk_v[2, sl], jnp.float32)
                sc = a1 + a2 + evg * wae
                lr = jnp.where(sc > 0.0, sc, sc * 0.01)
                el = jnp.exp(-lr)
                elem_v[sl] = el
                plsc.addupdate_scatter(rs_v, [si], el)
                plsc.addupdate_scatter(es_v, [si], el * evg)

        def chunk_body(i, _):
            pkld = pltpu.async_copy(pk_h.at[pkbase + i], pkA_v, sem3)
            spoff = base_e + jnp.maximum(i - 1, 0) * C
            spld = pltpu.async_copy(src_h.at[pl.ds(spoff, C)], sidxp_v, sem4)
            pkld.wait()
            gather = pltpu.async_copy(feat_h.at[pkA_v.at[1]], rowsA_v, sem)
            spld.wait()
            scat = pltpu.async_copy(rowsB_v, acc_sh.at[sidxp_v], sem2,
                                    add=True)
            compute_elem(pkA_v)
            gather.wait()
            scat.wait()
            for r in range(C):
                b = plsc.load_gather(elem_v, [jnp.full((L,), r, jnp.int32)])
                for k in range(128 // L):
                    sl2 = pl.ds(k * L, L)
                    rowsB_v[r, sl2] = rowsA_v[r, sl2] * b
            return 0

        lax.fori_loop(0, NCH, chunk_body, 0)
        poff = pl.multiple_of((c * NS + s) * NPS, 8)
        pltpu.sync_copy(rs_v, rsp_h.at[pl.ds(poff, NPS)])
        pltpu.sync_copy(es_v, esp_h.at[pl.ds(poff, NPS)])
        pltpu.sync_copy(src_h.at[pl.ds(base_e + (NCH - 1) * C, C)], sidxp_v)
        pltpu.sync_copy(rowsB_v, acc_sh.at[sidxp_v], add=True)
        plsc.subcore_barrier()

        def out_body(i, _):
            base = s * RPT + i * WC
            pltpu.sync_copy(aridx_h.at[pl.ds(base, WC)], zidx_v)
            pltpu.sync_copy(acc_sh.at[zidx_v], rowsA_v.at[pl.ds(0, WC)])
            pltpu.sync_copy(rowsA_v.at[pl.ds(0, WC)],
                            accd_h.at[pl.ds(hoff + base, WC)])
            return 0

        lax.fori_loop(0, NZC, out_body, 0)

    return sc_kernel(src, pk, apk, wae16, featf, aridx, zerosf)


def _tc3_body(feat_ref, acc_ref, rspt_ref, espt_ref, out_ref):
    H = feat_ref.shape[0]
    pieces = []
    for h in range(H):
        rs = jnp.sum(rspt_ref[h], axis=-1)
        es = jnp.sum(espt_ref[h], axis=-1)
        rcp = 1.0 / rs
        pieces += [feat_ref[h], acc_ref[h] * rcp[:, None],
                   (es * rcp)[:, None]]
    out_ref[...] = jnp.concatenate(pieces, axis=1)


def _tc3_stage(feat, accd, rspt, espt, N):
    H, NP, D = accd.shape
    BS = N // 10 if N % 10 == 0 else N
    NB = N // BS
    return pl.pallas_call(
        _tc3_body,
        grid=(NB,),
        in_specs=[
            pl.BlockSpec((H, BS, D), lambda i: (0, i, 0)),
            pl.BlockSpec((H, BS, D), lambda i: (0, i, 0)),
            pl.BlockSpec((H, BS, NS), lambda i: (0, i, 0)),
            pl.BlockSpec((H, BS, NS), lambda i: (0, i, 0)),
        ],
        out_specs=pl.BlockSpec((BS, H * (2 * D + 1)), lambda i: (i, 0)),
        out_shape=jax.ShapeDtypeStruct((N, H * (2 * D + 1)), jnp.float32),
    )(feat, accd, rspt, espt)


def kernel(x, edge_index, edge_values, num_nodes, W1, b1, W2, b2, Wa, ba):
    N, D_IN = x.shape
    H = W1.shape[0]
    D_OUT = W2.shape[2]
    E = edge_index.shape[1]

    wsd = jnp.stack([Wa[:, :D_OUT, 0], Wa[:, D_OUT:2 * D_OUT, 0]], axis=-1) * 0.05
    bsd = jnp.concatenate([ba * 0.05, jnp.zeros_like(ba)], axis=1)[:, None, :]
    wae16 = jnp.repeat(Wa[:, 2 * D_OUT, 0] * 0.05, L).reshape(H * L)

    feat, a2 = _tc_stage(x, W1, b1, W2, b2, wsd, bsd)

    NP = -(-(N + 1) // (NS * WC)) * (NS * WC)
    NPS = -(-(N + 1) // 8) * 8
    featp = feat.reshape(H * N, D_OUT)
    ap = jnp.pad(a2, ((0, 0), (0, NPS - N), (0, 0)))
    asr16 = lax.bitcast_convert_type(
        ap[:, :, 0].astype(jnp.bfloat16), jnp.uint16).astype(jnp.uint32)
    adt16 = lax.bitcast_convert_type(
        ap[:, :, 1].astype(jnp.bfloat16), jnp.uint16).astype(jnp.uint32)
    apk = lax.bitcast_convert_type(
        (adt16 << 16) | asr16, jnp.float32).reshape(H * NPS)

    EPT = -(-E // (NS * C)) * C
    EPAD = EPT * NS - E
    src = edge_index[0]
    dst = edge_index[1]
    ev = edge_values
    if EPAD:
        src = jnp.concatenate([src, jnp.full((EPAD,), N, jnp.int32)])
        dst = jnp.concatenate([dst, jnp.zeros((EPAD,), jnp.int32)])
        ev = jnp.concatenate([ev, jnp.zeros((EPAD,), jnp.float32)])
    NCH = EPT // C
    dsta = dst[None, :] + (jnp.arange(H, dtype=jnp.int32) * N)[:, None]
    src_r = jnp.broadcast_to(src.reshape(1, NS, NCH, C), (H, NS, NCH, C))
    ev_r = jnp.broadcast_to(
        lax.bitcast_convert_type(ev, jnp.int32).reshape(1, NS, NCH, C),
        (H, NS, NCH, C))
    pk = jnp.stack([src_r, dsta.reshape(H, NS, NCH, C), ev_r],
                   axis=3).reshape(H * NS * NCH, 3, C)

    aridx = jnp.arange(NP, dtype=jnp.int32)
    zerosf = jnp.zeros((NPS,), jnp.float32)
    accd, rsp, esp = _sc_stage(src, pk, apk, wae16, featp,
                               aridx, zerosf, NP, NPS, D_OUT, EPT)
    rspt = rsp.reshape(H, NS, NPS).transpose(0, 2, 1)
    espt = esp.reshape(H, NS, NPS).transpose(0, 2, 1)
    return _tc3_stage(feat, accd.reshape(H, NP, D_OUT), rspt, espt, N)
